# Initial kernel scaffold; baseline (speedup 1.0000x reference)
import jax
import jax.numpy as jnp
from jax.experimental import pallas as pl


def kernel(rays_o, rays_d, z_vals, weights):
    raise NotImplementedError("write your pallas kernel here")


# speedup vs baseline: 247.5162x; 247.5162x over previous
import functools

import jax
import jax.numpy as jnp
from jax import lax
from jax.experimental import pallas as pl
from jax.experimental.pallas import tpu as pltpu
from jax.experimental.pallas import tpu_sc as plsc

RAYS = 65536
NSAMP = 64
NIMP = 128
NALL = NSAMP + NIMP
NWORK = 32
RPW = RAYS // NWORK
CH = 8
NCHUNK = RPW // CH

_f32 = jnp.float32
_i32 = jnp.int32


def _body(ro_h, rd_h, z_h, w_h, pts_h, za_h, zs_h,
          zin, win, oin, din, cf, bins, hist, samp, zall, ptsb):
    nc = 2
    wid = lax.axis_index("s") * nc + lax.axis_index("c")

    iota = lax.iota(_i32, 16)
    iota_f = iota.astype(_f32)
    ones_i = jnp.full((16,), 1, _i32)
    zero_i = jnp.full((16,), 0, _i32)

    def ray_body(r, _):
        rz = r * 64
        w0 = win[pl.ds(rz, 16)]
        w1 = win[pl.ds(rz + 16, 16)]
        w2 = win[pl.ds(rz + 32, 16)]
        w3 = win[pl.ds(rz + 48, 16)]
        eps = _f32(1e-5)
        a0 = jnp.where(iota >= 1, w0 + eps, _f32(0.0))
        a1 = w1 + eps
        a2 = w2 + eps
        a3 = jnp.where(iota <= 14, w3 + eps, _f32(0.0))
        c0 = plsc.cumsum(a0)
        c1 = plsc.cumsum(a1) + c0[15]
        c2 = plsc.cumsum(a2) + c1[15]
        c3 = plsc.cumsum(a3) + c2[15]
        total = c3[15]
        cf[pl.ds(0, 16)] = c0
        cf[pl.ds(16, 16)] = c1
        cf[pl.ds(32, 16)] = c2
        cf[pl.ds(48, 16)] = c3

        z0 = zin[pl.ds(rz, 16)]
        z1 = zin[pl.ds(rz + 16, 16)]
        z2 = zin[pl.ds(rz + 32, 16)]
        z3 = zin[pl.ds(rz + 48, 16)]
        zs0 = zin[pl.ds(rz + 1, 16)]
        zs1 = zin[pl.ds(rz + 17, 16)]
        zs2 = zin[pl.ds(rz + 33, 16)]
        zs3 = zin[pl.ds(rz + 49, 16)]
        half = _f32(0.5)
        bins[pl.ds(0, 16)] = half * (z0 + zs0)
        bins[pl.ds(16, 16)] = half * (z1 + zs1)
        bins[pl.ds(32, 16)] = half * (z2 + zs2)
        bins[pl.ds(48, 16)] = half * (z3 + zs3)

        for i in range(9):
            hist[pl.ds(16 * i, 16)] = zero_i
        tot_v = jnp.full((16,), 1.0, _f32) * total
        inv_v = _f32(1.0) / tot_v
        scale = _f32(127.0) * inv_v
        for i, cv in enumerate((c0, c1, c2, c3)):
            f = cv * scale
            ti = f.astype(_i32)
            m = ti + jnp.where(ti.astype(_f32) < f, 1, 0)
            m = jnp.minimum(m, 129)
            if i == 0:
                pass
            if i == 3:
                m = jnp.where(iota <= 14, m, 129)
            plsc.addupdate_scatter(hist, [m], ones_i)

        td = total * _f32(1.0 / 127.0)
        eps_t = _f32(1e-5) * total
        rs = r * 128
        carry = _i32(0)
        for i in range(8):
            h = hist[pl.ds(16 * i, 16)]
            inds = plsc.cumsum(h) + carry
            carry = inds[15]
            below = inds - 1
            above = jnp.minimum(inds, 62)
            cb = plsc.load_gather(cf, [below])
            ca = plsc.load_gather(cf, [above])
            bb = plsc.load_gather(bins, [below])
            ba = plsc.load_gather(bins, [above])
            u = (iota_f + _f32(16 * i)) * td
            denom = ca - cb
            rden = jnp.where(denom < eps_t, inv_v, _f32(1.0) / denom)
            t = (u - cb) * rden
            samp[pl.ds(rs + 16 * i, 16)] = bb + t * (ba - bb)

        for i in range(9):
            hist[pl.ds(16 * i, 16)] = zero_i
        rza = r * 192
        for i, zv in enumerate((z0, z1, z2, z3)):
            s = zero_i
            for step in (128, 64, 32, 16, 8, 4, 2, 1):
                cand = s + step
                idx = jnp.minimum(cand - 1, 127)
                v = plsc.load_gather(samp, [rs + idx])
                ok = (cand <= 128) & (v < zv)
                s = jnp.where(ok, cand, s)
            p = iota + (16 * i) + s
            plsc.store_scatter(zall, [rza + p], zv)
            plsc.addupdate_scatter(hist, [s], ones_i)

        carry = _i32(0)
        for i in range(8):
            h = hist[pl.ds(16 * i, 16)]
            r1 = plsc.cumsum(h) + carry
            carry = r1[15]
            q = iota + (16 * i) + r1
            sv = samp[pl.ds(rs + 16 * i, 16)]
            plsc.store_scatter(zall, [rza + q], sv)

        r3 = r * 3
        ov = oin[pl.ds(r3, 16)]
        dv = din[pl.ds(r3, 16)]
        ox, oy, oz = ov[0], ov[1], ov[2]
        dx, dy, dz = dv[0], dv[1], dv[2]
        rp = r * 576
        for i in range(12):
            zv = zall[pl.ds(rza + 16 * i, 16)]
            mi = (iota + (16 * i)) * 3 + rp
            plsc.store_scatter(ptsb, [mi], ox + dx * zv)
            plsc.store_scatter(ptsb, [mi + 1], oy + dy * zv)
            plsc.store_scatter(ptsb, [mi + 2], oz + dz * zv)
        return _

    def chunk_body(ci, _):
        base = wid * RPW + ci * CH
        pltpu.sync_copy(z_h.at[pl.ds(base * 64, CH * 64)], zin.at[pl.ds(0, CH * 64)])
        pltpu.sync_copy(w_h.at[pl.ds(base * 64, CH * 64)], win)
        pltpu.sync_copy(ro_h.at[pl.ds(base * 3, CH * 3)], oin.at[pl.ds(0, CH * 3)])
        pltpu.sync_copy(rd_h.at[pl.ds(base * 3, CH * 3)], din.at[pl.ds(0, CH * 3)])
        lax.fori_loop(0, CH, ray_body, 0)
        pltpu.sync_copy(samp, zs_h.at[pl.ds(base * 128, CH * 128)])
        pltpu.sync_copy(zall, za_h.at[pl.ds(base * 192, CH * 192)])
        pltpu.sync_copy(ptsb, pts_h.at[pl.ds(base * 576, CH * 576)])
        return _

    lax.fori_loop(0, NCHUNK, chunk_body, 0)


@functools.lru_cache(maxsize=1)
def _make_sc_call():
    mesh = plsc.VectorSubcoreMesh(
        core_axis_name="c", subcore_axis_name="s",
        num_cores=2, num_subcores=16)
    return pl.kernel(
        _body,
        out_type=[
            jax.ShapeDtypeStruct((RAYS * NALL * 3,), _f32),
            jax.ShapeDtypeStruct((RAYS * NALL,), _f32),
            jax.ShapeDtypeStruct((RAYS * NIMP,), _f32),
        ],
        mesh=mesh,
        compiler_params=pltpu.CompilerParams(needs_layout_passes=False),
        scratch_types=[
            pltpu.VMEM((CH * 64 + 16,), _f32),
            pltpu.VMEM((CH * 64,), _f32),
            pltpu.VMEM((CH * 3 + 16,), _f32),
            pltpu.VMEM((CH * 3 + 16,), _f32),
            pltpu.VMEM((64,), _f32),
            pltpu.VMEM((64,), _f32),
            pltpu.VMEM((144,), _i32),
            pltpu.VMEM((CH * 128,), _f32),
            pltpu.VMEM((CH * 192,), _f32),
            pltpu.VMEM((CH * 576,), _f32),
        ],
    )


@jax.jit
def kernel(rays_o, rays_d, z_vals, weights):
    pts_f, za_f, zs_f = _make_sc_call()(
        rays_o.reshape(-1), rays_d.reshape(-1),
        z_vals.reshape(-1), weights.reshape(-1))
    return (pts_f.reshape(RAYS, NALL, 3),
            za_f.reshape(RAYS, NALL),
            zs_f.reshape(RAYS, NIMP))
